# baseline (speedup 1.0000x reference)
import functools
import math

import jax
import jax.numpy as jnp
from jax import lax
from jax.experimental import pallas as pl
from jax.experimental.pallas import tpu as pltpu

_BM1 = 256
_BM2 = 1024
_F8 = jnp.float8_e4m3fn
_BF16 = jnp.bfloat16


def _col_stats(y, valid_rows, cs_ref, mx_ref, mn_ref, first):
    rows = lax.broadcasted_iota(jnp.int32, (y.shape[0], 1), 0)
    ok = rows < valid_rows
    cs = jnp.sum(jnp.where(ok, y, 0.0), axis=0, keepdims=True)
    mx = jnp.max(jnp.where(ok, y, -jnp.inf), axis=0, keepdims=True)
    mn = jnp.min(jnp.where(ok, y, jnp.inf), axis=0, keepdims=True)

    @pl.when(first)
    def _():
        cs_ref[...] = cs
        mx_ref[...] = mx
        mn_ref[...] = mn

    @pl.when(jnp.logical_not(first))
    def _():
        cs_ref[...] += cs
        mx_ref[...] = jnp.maximum(mx_ref[...], mx)
        mn_ref[...] = jnp.minimum(mn_ref[...], mn)


def _l1_body(a_ref, x_ref, b_ref, y_ref, aq_ref, rs_ref, cs_ref, mx_ref,
             mn_ref, *, n):
    i = pl.program_id(0)
    a = a_ref[...]
    aq_ref[...] = a.astype(_F8)
    rs_ref[...] = jnp.sum(a, axis=1, keepdims=True)
    y = jnp.dot(a, x_ref[...], preferred_element_type=jnp.float32)
    y = y + b_ref[...]
    y_ref[...] = y.astype(_BF16)
    _col_stats(y, n - i * _BM1, cs_ref, mx_ref, mn_ref, i == 0)


def _layer1(adj, fea, b, npad):
    n, d = fea.shape
    grid = (npad // _BM1,)
    one = lambda i: (0, 0)
    return pl.pallas_call(
        functools.partial(_l1_body, n=n),
        grid=grid,
        in_specs=[
            pl.BlockSpec((_BM1, n), lambda i: (i, 0)),
            pl.BlockSpec((n, d), one),
            pl.BlockSpec((1, d), one),
        ],
        out_specs=[
            pl.BlockSpec((_BM1, d), lambda i: (i, 0)),
            pl.BlockSpec((_BM1, n), lambda i: (i, 0)),
            pl.BlockSpec((_BM1, 1), lambda i: (i, 0)),
            pl.BlockSpec((1, d), one),
            pl.BlockSpec((1, d), one),
            pl.BlockSpec((1, d), one),
        ],
        out_shape=[
            jax.ShapeDtypeStruct((npad, d), _BF16),
            jax.ShapeDtypeStruct((npad, n), _F8),
            jax.ShapeDtypeStruct((npad, 1), jnp.float32),
            jax.ShapeDtypeStruct((1, d), jnp.float32),
            jax.ShapeDtypeStruct((1, d), jnp.float32),
            jax.ShapeDtypeStruct((1, d), jnp.float32),
        ],
    )(adj, fea, b)


def _l23_body(aq_ref, x1_ref, rs_ref, cs1_ref, mx1_ref, mn1_ref, b1_ref,
              b2_ref, fea_ref, out_ref, x1q_s, x2_s, x2q_s, cs2_s,
              mx2_s, mn2_s, *, n):
    p = pl.program_id(0)
    i = pl.program_id(1)
    inv_n = 1.0 / n

    def scale_of(cs, mx, mn):
        m = cs * inv_n
        s = jnp.max(jnp.maximum(mx - m, m - mn)) * (1.0 / 384.0)
        return m, jnp.maximum(s, 1e-30)

    m1, s1 = scale_of(cs1_ref[...], mx1_ref[...], mn1_ref[...])

    @pl.when((p == 0) & (i == 0))
    def _():
        x1q_s[...] = ((x1_ref[pl.ds(0, n), :].astype(jnp.float32) - m1) *
                      (1.0 / s1)).astype(_F8)

    @pl.when(p == 0)
    def _():
        y = jnp.dot(aq_ref[...], x1q_s[...],
                    preferred_element_type=jnp.float32)
        y = y * s1 + rs_ref[...] * m1 + b1_ref[...]
        x2_s[pl.ds(i * _BM2, _BM2), :] = y.astype(_BF16)
        _col_stats(y, n - i * _BM2, cs2_s, mx2_s, mn2_s, i == 0)

    m2, s2 = scale_of(cs2_s[...], mx2_s[...], mn2_s[...])

    @pl.when((p == 1) & (i == 0))
    def _():
        x2q_s[...] = ((x2_s[pl.ds(0, n), :].astype(jnp.float32) - m2) *
                      (1.0 / s2)).astype(_F8)

    @pl.when(p == 1)
    def _():
        y = jnp.dot(aq_ref[...], x2q_s[...],
                    preferred_element_type=jnp.float32)
        y = y * s2 + rs_ref[...] * m2 + b2_ref[...]
        x1_blk = x1_ref[pl.ds(i * _BM2, _BM2), :].astype(jnp.float32)
        x2_blk = x2_s[pl.ds(i * _BM2, _BM2), :].astype(jnp.float32)
        out_ref[...] = (fea_ref[...] + x1_blk + x2_blk + y) * 0.25


def _layers23(adj_q, x1, rs, cs1, mx1, mn1, b1, b2, fea):
    n, d = fea.shape
    npad = adj_q.shape[0]
    grid = (2, npad // _BM2)
    one = lambda p, i: (0, 0)
    ph1 = lambda p, i: (i * p, 0)
    return pl.pallas_call(
        functools.partial(_l23_body, n=n),
        grid=grid,
        in_specs=[
            pl.BlockSpec((_BM2, n), lambda p, i: (i, 0)),
            pl.BlockSpec((npad, d), one),
            pl.BlockSpec((_BM2, 1), lambda p, i: (i, 0)),
            pl.BlockSpec((1, d), one),
            pl.BlockSpec((1, d), one),
            pl.BlockSpec((1, d), one),
            pl.BlockSpec((1, d), one),
            pl.BlockSpec((1, d), one),
            pl.BlockSpec((_BM2, d), ph1),
        ],
        out_specs=pl.BlockSpec((_BM2, d), ph1),
        out_shape=jax.ShapeDtypeStruct((n, d), jnp.float32),
        scratch_shapes=[
            pltpu.VMEM((n, d), _F8),
            pltpu.VMEM((npad, d), _BF16),
            pltpu.VMEM((n, d), _F8),
            pltpu.VMEM((1, d), jnp.float32),
            pltpu.VMEM((1, d), jnp.float32),
            pltpu.VMEM((1, d), jnp.float32),
        ],
    )(adj_q, x1, rs, cs1, mx1, mn1, b1, b2, fea)


def kernel(fea, adj, b0, b1, b2):
    n, d = fea.shape
    blk = math.lcm(_BM1, _BM2)
    npad = -(-n // blk) * blk
    x1, adj_q, rs, cs1, mx1, mn1 = _layer1(adj, fea, b0.reshape(1, d), npad)
    return _layers23(adj_q, x1, rs, cs1, mx1, mn1, b1.reshape(1, d),
                     b2.reshape(1, d), fea)


# speedup vs baseline: 1.0287x; 1.0287x over previous
import functools
import math

import jax
import jax.numpy as jnp
from jax import lax
from jax.experimental import pallas as pl
from jax.experimental.pallas import tpu as pltpu

_BM1 = 320
_BM2 = 1024
_F8 = jnp.float8_e4m3fn
_BF16 = jnp.bfloat16


def _col_stats(y, valid_rows, cs_ref, mx_ref, mn_ref, first):
    rows = lax.broadcasted_iota(jnp.int32, (y.shape[0], 1), 0)
    ok = rows < valid_rows
    cs = jnp.sum(jnp.where(ok, y, 0.0), axis=0, keepdims=True)
    mx = jnp.max(jnp.where(ok, y, -jnp.inf), axis=0, keepdims=True)
    mn = jnp.min(jnp.where(ok, y, jnp.inf), axis=0, keepdims=True)

    @pl.when(first)
    def _():
        cs_ref[...] = cs
        mx_ref[...] = mx
        mn_ref[...] = mn

    @pl.when(jnp.logical_not(first))
    def _():
        cs_ref[...] += cs
        mx_ref[...] = jnp.maximum(mx_ref[...], mx)
        mn_ref[...] = jnp.minimum(mn_ref[...], mn)


def _l1_body(a_ref, x_ref, b_ref, y_ref, aq_ref, rs_ref, cs_ref, mx_ref,
             mn_ref, *, n):
    i = pl.program_id(0)
    a = a_ref[...]
    aq_ref[...] = a.astype(_F8)
    rs_ref[...] = jnp.sum(a, axis=1, keepdims=True)
    y = jnp.dot(a, x_ref[...], preferred_element_type=jnp.float32)
    y = y + b_ref[...]
    y_ref[...] = y.astype(_BF16)
    _col_stats(y, n - i * _BM1, cs_ref, mx_ref, mn_ref, i == 0)


def _layer1(adj, fea, b, npad):
    n, d = fea.shape
    grid = (npad // _BM1,)
    one = lambda i: (0, 0)
    return pl.pallas_call(
        functools.partial(_l1_body, n=n),
        grid=grid,
        in_specs=[
            pl.BlockSpec((_BM1, n), lambda i: (i, 0)),
            pl.BlockSpec((n, d), one),
            pl.BlockSpec((1, d), one),
        ],
        out_specs=[
            pl.BlockSpec((_BM1, d), lambda i: (i, 0)),
            pl.BlockSpec((_BM1, n), lambda i: (i, 0)),
            pl.BlockSpec((_BM1, 1), lambda i: (i, 0)),
            pl.BlockSpec((1, d), one),
            pl.BlockSpec((1, d), one),
            pl.BlockSpec((1, d), one),
        ],
        out_shape=[
            jax.ShapeDtypeStruct((npad, d), _BF16),
            jax.ShapeDtypeStruct((npad, n), _F8),
            jax.ShapeDtypeStruct((npad, 1), jnp.float32),
            jax.ShapeDtypeStruct((1, d), jnp.float32),
            jax.ShapeDtypeStruct((1, d), jnp.float32),
            jax.ShapeDtypeStruct((1, d), jnp.float32),
        ],
    )(adj, fea, b)


def _l23_body(aq_ref, x1_ref, rs_ref, cs1_ref, mx1_ref, mn1_ref, b1_ref,
              b2_ref, fea_ref, out_ref, x1q_s, x2_s, x2q_s, cs2_s,
              mx2_s, mn2_s, *, n):
    p = pl.program_id(0)
    i = pl.program_id(1)
    inv_n = 1.0 / n

    def scale_of(cs, mx, mn):
        m = cs * inv_n
        s = jnp.max(jnp.maximum(mx - m, m - mn)) * (1.0 / 384.0)
        return m, jnp.maximum(s, 1e-30)

    m1, s1 = scale_of(cs1_ref[...], mx1_ref[...], mn1_ref[...])

    @pl.when((p == 0) & (i == 0))
    def _():
        x1q_s[...] = ((x1_ref[pl.ds(0, n), :].astype(jnp.float32) - m1) *
                      (1.0 / s1)).astype(_F8)

    @pl.when(p == 0)
    def _():
        y = jnp.dot(aq_ref[...], x1q_s[...],
                    preferred_element_type=jnp.float32)
        y = y * s1 + rs_ref[...] * m1 + b1_ref[...]
        x2_s[pl.ds(i * _BM2, _BM2), :] = y.astype(_BF16)
        _col_stats(y, n - i * _BM2, cs2_s, mx2_s, mn2_s, i == 0)

    m2, s2 = scale_of(cs2_s[...], mx2_s[...], mn2_s[...])

    @pl.when((p == 1) & (i == 0))
    def _():
        x2q_s[...] = ((x2_s[pl.ds(0, n), :].astype(jnp.float32) - m2) *
                      (1.0 / s2)).astype(_F8)

    @pl.when(p == 1)
    def _():
        g = pl.num_programs(1)
        j = lax.rem(i + g - 1, g)
        y = jnp.dot(aq_ref[...], x2q_s[...],
                    preferred_element_type=jnp.float32)
        y = y * s2 + rs_ref[...] * m2 + b2_ref[...]
        x1_blk = x1_ref[pl.ds(j * _BM2, _BM2), :].astype(jnp.float32)
        x2_blk = x2_s[pl.ds(j * _BM2, _BM2), :].astype(jnp.float32)
        out_ref[...] = (fea_ref[...] + x1_blk + x2_blk + y) * 0.25


def _layers23(adj_q, x1, rs, cs1, mx1, mn1, b1, b2, fea):
    n, d = fea.shape
    npad = adj_q.shape[0]
    g = npad // _BM2
    grid = (2, g)
    one = lambda p, i: (0, 0)
    rot = lambda p, i: ((1 - p) * i + p * ((i + g - 1) % g), 0)
    ph1 = lambda p, i: (p * ((i + g - 1) % g), 0)
    return pl.pallas_call(
        functools.partial(_l23_body, n=n),
        grid=grid,
        in_specs=[
            pl.BlockSpec((_BM2, n), rot),
            pl.BlockSpec((npad, d), one),
            pl.BlockSpec((_BM2, 1), rot),
            pl.BlockSpec((1, d), one),
            pl.BlockSpec((1, d), one),
            pl.BlockSpec((1, d), one),
            pl.BlockSpec((1, d), one),
            pl.BlockSpec((1, d), one),
            pl.BlockSpec((_BM2, d), ph1),
        ],
        out_specs=pl.BlockSpec((_BM2, d), ph1),
        out_shape=jax.ShapeDtypeStruct((n, d), jnp.float32),
        scratch_shapes=[
            pltpu.VMEM((n, d), _F8),
            pltpu.VMEM((npad, d), _BF16),
            pltpu.VMEM((n, d), _F8),
            pltpu.VMEM((1, d), jnp.float32),
            pltpu.VMEM((1, d), jnp.float32),
            pltpu.VMEM((1, d), jnp.float32),
        ],
    )(adj_q, x1, rs, cs1, mx1, mn1, b1, b2, fea)


def kernel(fea, adj, b0, b1, b2):
    n, d = fea.shape
    blk = math.lcm(_BM1, _BM2)
    npad = -(-n // blk) * blk
    x1, adj_q, rs, cs1, mx1, mn1 = _layer1(adj, fea, b0.reshape(1, d), npad)
    return _layers23(adj_q, x1, rs, cs1, mx1, mn1, b1.reshape(1, d),
                     b2.reshape(1, d), fea)
